# baseline (speedup 1.0000x reference)
import functools

import jax
import jax.numpy as jnp
from jax import lax
from jax.experimental import pallas as pl
from jax.experimental.pallas import tpu as pltpu
from jax.experimental.pallas import tpu_sc as plsc

_N0 = 576
_N = _N0 * _N0
_H = 32
_ROWS = 2209
_RPAD = 2304
_NC, _NS, _L = 2, 16, 16
_HG = 8
_NQ = 8
_QROWS = _N0 // _NQ
_R = 8
_NSUB = _QROWS // _R
_NV = _N0 // _L


def _make_kernel():
    mesh = plsc.VectorSubcoreMesh(core_axis_name="c", subcore_axis_name="s")

    @functools.partial(
        pl.kernel,
        mesh=mesh,
        out_type=jax.ShapeDtypeStruct((_H * _N0, _N0), jnp.float32),
        scratch_types=[
            pltpu.VMEM((_HG * _RPAD,), jnp.float32),
            pltpu.VMEM((2, _R, _N0), jnp.int32),
            pltpu.VMEM((2, _HG, _R, _N0), jnp.float32),
            pltpu.SemaphoreType.DMA,
            pltpu.SemaphoreType.DMA,
            pltpu.SemaphoreType.DMA,
            pltpu.SemaphoreType.DMA,
            pltpu.SemaphoreType.DMA,
        ],
        compiler_params=pltpu.CompilerParams(
            use_tc_tiling_on_sc=True, needs_layout_passes=False
        ),
    )
    def k(table_hbm, idx_hbm, out_hbm, trows_v, idx_v, out_v,
          sem_t, sem_i0, sem_i1, sem_o0, sem_o1):
        wid = lax.axis_index("s") * _NC + lax.axis_index("c")
        hblk = wid % (_H // _HG)
        quar = wid // (_H // _HG)
        sem_i = (sem_i0, sem_i1)
        sem_o = (sem_o0, sem_o1)

        cp_t = pltpu.make_async_copy(
            table_hbm.at[pl.ds(hblk * _HG * _RPAD, _HG * _RPAD)], trows_v, sem_t
        )
        cp_t.start()

        idx_row0 = quar * _QROWS

        def start_idx(s, buf):
            pltpu.make_async_copy(
                idx_hbm.at[pl.ds(idx_row0 + s * _R, _R), :],
                idx_v.at[buf],
                sem_i[buf],
            ).start()

        def wait_idx(buf):
            pltpu.make_async_copy(
                idx_hbm.at[pl.ds(0, _R), :], idx_v.at[buf], sem_i[buf]
            ).wait()

        def wait_out(buf):
            for hl in range(_HG):
                pltpu.make_async_copy(
                    out_v.at[buf, hl], out_hbm.at[pl.ds(0, _R), :], sem_o[buf]
                ).wait()

        start_idx(0, 0)
        cp_t.wait()

        def gather_round(s, buf):
            @pl.when(s < _NSUB - 1)
            def _():
                start_idx(s + 1, 1 - buf)

            wait_idx(buf)

            @pl.when(s >= 2)
            def _():
                wait_out(buf)

            @plsc.parallel_loop(0, _R)
            def _(r):
                @plsc.parallel_loop(0, _NV, unroll=2)
                def _(c):
                    iv = idx_v[buf, r, pl.ds(c * _L, _L)]
                    for hl in range(_HG):
                        out_v[buf, hl, r, pl.ds(c * _L, _L)] = plsc.load_gather(
                            trows_v, [iv + hl * _RPAD]
                        )

            for hl in range(_HG):
                row0 = (hblk * _HG + hl) * _N0 + quar * _QROWS + s * _R
                pltpu.make_async_copy(
                    out_v.at[buf, hl],
                    out_hbm.at[pl.ds(row0, _R), :],
                    sem_o[buf],
                ).start()

        def body(s, carry):
            @pl.when(s % 2 == 0)
            def _():
                gather_round(s, 0)

            @pl.when(s % 2 == 1)
            def _():
                gather_round(s, 1)

            return carry

        lax.fori_loop(0, _NSUB, body, 0)
        wait_out(0)
        wait_out(1)

    return k


_gather_kernel = _make_kernel()


def kernel(relative_position_bias_table, relative_position_index):
    table_rows = jnp.pad(
        relative_position_bias_table.T, ((0, 0), (0, _RPAD - _ROWS))
    )
    table_flat = table_rows.reshape(-1)
    idx2d = relative_position_index.astype(jnp.int32)
    out = _gather_kernel(table_flat, idx2d)
    n0, n1 = relative_position_index.shape
    return out.reshape(_H, n0, n1)
